# baseline (speedup 1.0000x reference)
import jax
import jax.numpy as jnp
from jax import lax
from jax.experimental import pallas as pl
from jax.experimental.pallas import tpu as pltpu
from jax.experimental.pallas import tpu_sc as plsc

_B = 2048
_IN = 4096
_OUT = 8192
_L = 16

_NC = 2
_NS = 16
_NW = _NC * _NS
_ROWS_W = _B // _NW
_RB = 8
_NRB = _ROWS_W // _RB
_C = 512
_NCH = _OUT // _C
_G = _C // _L

_MT = [
    [0, 0, 0, 0, 0, 0, 0, 0, 1, 1, 1, 1, 1, 1, 1, 1],
    [0, 0, 1, 1, 0, 0, 1, 1, -1, -1, 0, 0, -1, -1, 0, 0],
    [0, 0, 0, 0, 1, 1, 1, 1, -1, -1, -1, -1, 0, 0, 0, 0],
    [0, 1, -1, 0, -1, 0, -2, -1, 1, 2, 0, 1, 0, 1, -1, 0],
]


def _coeff_body(w_ref, mt_ref, out_ref):
    w = w_ref[...]
    m = jnp.max(w, axis=-1, keepdims=True)
    e = jnp.exp(w - m)
    p = e / jnp.sum(e, axis=-1, keepdims=True)
    out_ref[...] = lax.dot_general(
        mt_ref[...], p, (((1,), (1,)), ((), ())),
        preferred_element_type=jnp.float32,
        precision=lax.Precision.HIGHEST,
    )


def _sc_body(x_hbm, iab_hbm, pos_hbm, c0_hbm, ca_hbm, cb_hbm, cab_hbm,
             out_hbm, xbuf, iabbuf, posbuf, c0buf, cabuf, cbbuf, cabbuf,
             obuf, xsem, osems):
    wid = lax.axis_index("s") * _NC + lax.axis_index("c")
    r0 = wid * _ROWS_W

    for r in range(_RB):
        pltpu.async_copy(
            x_hbm.at[r0 + r], xbuf.at[pl.ds(r * _IN, _IN)], xsem)
    pltpu.sync_copy(iab_hbm, iabbuf)
    pltpu.sync_copy(pos_hbm, posbuf)
    pltpu.sync_copy(c0_hbm, c0buf)
    pltpu.sync_copy(ca_hbm, cabuf)
    pltpu.sync_copy(cb_hbm, cbbuf)
    pltpu.sync_copy(cab_hbm, cabbuf)

    def rb_body(rb, carry0):
        xslot = lax.rem(rb, 2)
        row = r0 + rb * _RB
        xofs = xslot * (_RB * _IN)
        for r in range(_RB):
            pltpu.make_async_copy(
                x_hbm.at[row + r],
                xbuf.at[pl.ds(xofs + r * _IN, _IN)], xsem).wait()

        @pl.when(rb < _NRB - 1)
        def _():
            nofs = (1 - xslot) * (_RB * _IN)
            for r in range(_RB):
                pltpu.async_copy(
                    x_hbm.at[row + _RB + r],
                    xbuf.at[pl.ds(nofs + r * _IN, _IN)], xsem)

        def jc_body(jc, carry1):
            oslot = lax.rem(jc, 2)
            j0 = jc * _C

            oofs = oslot * (_RB * _C)

            @pl.when(rb * _NCH + jc >= 2)
            def _():
                for r in range(_RB):
                    pltpu.make_async_copy(
                        obuf.at[pl.ds(oofs + r * _C, _C)],
                        out_hbm.at[row + r, pl.ds(j0, _C)],
                        osems.at[oslot]).wait()

            @plsc.parallel_loop(0, _G, unroll=4)
            def g_body(g):
                gofs = j0 + g * _L
                iab = iabbuf[pl.ds(gofs, _L)]
                ia = jnp.bitwise_and(iab, 4095)
                ib = jnp.right_shift(iab, 12)
                posv = posbuf[pl.ds(gofs, _L)]
                c0 = c0buf[pl.ds(gofs, _L)]
                ca = cabuf[pl.ds(gofs, _L)]
                cb = cbbuf[pl.ds(gofs, _L)]
                cab = cabbuf[pl.ds(gofs, _L)]
                for r in range(_RB):
                    xrow = xbuf.at[pl.ds(xofs + r * _IN, _IN)]
                    av = plsc.load_gather(xrow, [ia])
                    bv = plsc.load_gather(xrow, [ib])
                    o = c0 + ca * av + cb * bv + cab * (av * bv)
                    plsc.store_scatter(obuf, [posv + (oofs + r * _C)], o)

            for r in range(_RB):
                pltpu.async_copy(
                    obuf.at[pl.ds(oofs + r * _C, _C)],
                    out_hbm.at[row + r, pl.ds(j0, _C)],
                    osems.at[oslot])
            return carry1

        lax.fori_loop(0, _NCH, jc_body, 0)
        return carry0

    lax.fori_loop(0, _NRB, rb_body, 0)

    for oslot in range(2):
        for r in range(_RB):
            pltpu.make_async_copy(
                obuf.at[pl.ds(oslot * (_RB * _C) + r * _C, _C)],
                out_hbm.at[r0 + r, pl.ds(0, _C)],
                osems.at[oslot]).wait()


def kernel(x, weights, idx_a, idx_b):
    coeffs = pl.pallas_call(
        _coeff_body,
        out_shape=jax.ShapeDtypeStruct((4, _OUT), jnp.float32),
    )(weights, jnp.asarray(_MT, dtype=jnp.float32))
    c0, ca, cb, cab = coeffs[0], coeffs[1], coeffs[2], coeffs[3]
    iab = idx_a + (idx_b << 12)

    bank = jnp.bitwise_and(idx_a, 15).reshape(_NCH, _C)
    order = jnp.argsort(bank, axis=-1, stable=True)
    deal = order.reshape(_NCH, _L, _G).transpose(0, 2, 1).reshape(_NCH, _C)
    pos = deal.reshape(-1).astype(jnp.int32)
    permg = (deal + (jnp.arange(_NCH, dtype=jnp.int32) * _C)[:, None]
             ).reshape(-1)
    iab = iab[permg]
    c0, ca, cb, cab = c0[permg], ca[permg], cb[permg], cab[permg]

    sc = pl.kernel(
        _sc_body,
        out_type=jax.ShapeDtypeStruct((_B, _OUT), jnp.float32),
        mesh=plsc.VectorSubcoreMesh(core_axis_name="c", subcore_axis_name="s"),
        compiler_params=pltpu.CompilerParams(needs_layout_passes=False),
        scratch_types=[
            pltpu.VMEM((2 * _RB * _IN,), jnp.float32),
            pltpu.VMEM((_OUT,), jnp.int32),
            pltpu.VMEM((_OUT,), jnp.int32),
            pltpu.VMEM((_OUT,), jnp.float32),
            pltpu.VMEM((_OUT,), jnp.float32),
            pltpu.VMEM((_OUT,), jnp.float32),
            pltpu.VMEM((_OUT,), jnp.float32),
            pltpu.VMEM((2 * _RB * _C,), jnp.float32),
            pltpu.SemaphoreType.DMA,
            pltpu.SemaphoreType.DMA((2,)),
        ],
    )
    return sc(x, iab, pos, c0, ca, cb, cab)


# speedup vs baseline: 1.0959x; 1.0959x over previous
import jax
import jax.numpy as jnp
from jax import lax
from jax.experimental import pallas as pl
from jax.experimental.pallas import tpu as pltpu
from jax.experimental.pallas import tpu_sc as plsc

_B = 2048
_IN = 4096
_OUT = 8192
_L = 16

_NC = 2
_NS = 16
_NW = _NC * _NS
_ROWS_W = _B // _NW
_RB = 8
_NRB = _ROWS_W // _RB
_C = 1024
_NCH = _OUT // _C
_G = _C // _L
_T = 6 * _C

_MT = [
    [0, 0, 0, 0, 0, 0, 0, 0, 1, 1, 1, 1, 1, 1, 1, 1],
    [0, 0, 1, 1, 0, 0, 1, 1, -1, -1, 0, 0, -1, -1, 0, 0],
    [0, 0, 0, 0, 1, 1, 1, 1, -1, -1, -1, -1, 0, 0, 0, 0],
    [0, 1, -1, 0, -1, 0, -2, -1, 1, 2, 0, 1, 0, 1, -1, 0],
]


def _coeff_body(w_ref, mt_ref, out_ref):
    w = w_ref[...]
    m = jnp.max(w, axis=-1, keepdims=True)
    e = jnp.exp(w - m)
    p = e / jnp.sum(e, axis=-1, keepdims=True)
    out_ref[...] = lax.dot_general(
        mt_ref[...], p, (((1,), (1,)), ((), ())),
        preferred_element_type=jnp.float32,
        precision=lax.Precision.HIGHEST,
    )


def _sc_body(x_hbm, tab_hbm, out_hbm, xbuf, tbuf, obuf, xsem, tsem, osems):
    wid = lax.axis_index("s") * _NC + lax.axis_index("c")
    r0 = wid * _ROWS_W

    for r in range(_RB):
        pltpu.async_copy(
            x_hbm.at[r0 + r], xbuf.at[pl.ds(r * _IN, _IN)], xsem)
    pltpu.async_copy(tab_hbm.at[pl.ds(0, _T)], tbuf.at[pl.ds(0, _T)], tsem)

    def rb_body(rb, carry0):
        xslot = lax.rem(rb, 2)
        row = r0 + rb * _RB
        xofs = xslot * (_RB * _IN)
        for r in range(_RB):
            pltpu.make_async_copy(
                x_hbm.at[row + r],
                xbuf.at[pl.ds(xofs + r * _IN, _IN)], xsem).wait()

        @pl.when(rb < _NRB - 1)
        def _():
            nofs = (1 - xslot) * (_RB * _IN)
            for r in range(_RB):
                pltpu.async_copy(
                    x_hbm.at[row + _RB + r],
                    xbuf.at[pl.ds(nofs + r * _IN, _IN)], xsem)

        def jc_body(jc, carry1):
            n = rb * _NCH + jc
            tslot = lax.rem(n, 2)
            tofs = tslot * _T
            oslot = lax.rem(jc, 2)
            oofs = oslot * (_RB * _C)
            j0 = jc * _C

            pltpu.make_async_copy(
                tab_hbm.at[pl.ds(jc * _T, _T)],
                tbuf.at[pl.ds(tofs, _T)], tsem).wait()

            @pl.when(n < _NRB * _NCH - 1)
            def _():
                nj = lax.rem(jc + 1, _NCH)
                pltpu.async_copy(
                    tab_hbm.at[pl.ds(nj * _T, _T)],
                    tbuf.at[pl.ds((1 - tslot) * _T, _T)], tsem)

            @pl.when(n >= 2)
            def _():
                pltpu.make_async_copy(
                    obuf.at[pl.ds(oofs, _RB * _C)],
                    out_hbm.at[row], osems.at[oslot]).wait()

            @plsc.parallel_loop(0, _G, unroll=4)
            def g_body(g):
                gofs = tofs + g * _L
                iab = tbuf[pl.ds(gofs, _L)]
                ia = jnp.bitwise_and(iab, 4095)
                ib = jnp.right_shift(iab, 12)
                posv = tbuf[pl.ds(gofs + _C, _L)]
                c0 = plsc.bitcast(tbuf[pl.ds(gofs + 2 * _C, _L)], jnp.float32)
                ca = plsc.bitcast(tbuf[pl.ds(gofs + 3 * _C, _L)], jnp.float32)
                cb = plsc.bitcast(tbuf[pl.ds(gofs + 4 * _C, _L)], jnp.float32)
                cab = plsc.bitcast(tbuf[pl.ds(gofs + 5 * _C, _L)], jnp.float32)
                for r in range(_RB):
                    xrow = xbuf.at[pl.ds(xofs + r * _IN, _IN)]
                    av = plsc.load_gather(xrow, [ia])
                    bv = plsc.load_gather(xrow, [ib])
                    o = c0 + ca * av + cb * bv + cab * (av * bv)
                    plsc.store_scatter(obuf, [posv + (oofs + r * _C)], o)

            for r in range(_RB):
                pltpu.async_copy(
                    obuf.at[pl.ds(oofs + r * _C, _C)],
                    out_hbm.at[row + r, pl.ds(j0, _C)],
                    osems.at[oslot])
            return carry1

        lax.fori_loop(0, _NCH, jc_body, 0)
        return carry0

    lax.fori_loop(0, _NRB, rb_body, 0)

    for oslot in range(2):
        pltpu.make_async_copy(
            obuf.at[pl.ds(oslot * (_RB * _C), _RB * _C)],
            out_hbm.at[r0], osems.at[oslot]).wait()


def kernel(x, weights, idx_a, idx_b):
    coeffs = pl.pallas_call(
        _coeff_body,
        out_shape=jax.ShapeDtypeStruct((4, _OUT), jnp.float32),
    )(weights, jnp.asarray(_MT, dtype=jnp.float32))
    c0, ca, cb, cab = coeffs[0], coeffs[1], coeffs[2], coeffs[3]
    iab = idx_a + (idx_b << 12)

    bank = jnp.bitwise_and(idx_a, 15).reshape(_NCH, _C)
    order = jnp.argsort(bank, axis=-1, stable=True)
    deal = order.reshape(_NCH, _L, _G).transpose(0, 2, 1).reshape(_NCH, _C)
    pos = deal.astype(jnp.int32)
    permg = (deal + (jnp.arange(_NCH, dtype=jnp.int32) * _C)[:, None]
             ).reshape(-1)

    def as_i32(v):
        return lax.bitcast_convert_type(v[permg].reshape(_NCH, _C), jnp.int32)

    tab = jnp.stack([
        iab[permg].reshape(_NCH, _C), pos,
        as_i32(c0), as_i32(ca), as_i32(cb), as_i32(cab),
    ], axis=1).reshape(-1)

    sc = pl.kernel(
        _sc_body,
        out_type=jax.ShapeDtypeStruct((_B, _OUT), jnp.float32),
        mesh=plsc.VectorSubcoreMesh(core_axis_name="c", subcore_axis_name="s"),
        compiler_params=pltpu.CompilerParams(needs_layout_passes=False),
        scratch_types=[
            pltpu.VMEM((2 * _RB * _IN,), jnp.float32),
            pltpu.VMEM((2 * _T,), jnp.int32),
            pltpu.VMEM((2 * _RB * _C,), jnp.float32),
            pltpu.SemaphoreType.DMA,
            pltpu.SemaphoreType.DMA,
            pltpu.SemaphoreType.DMA((2,)),
        ],
    )
    return sc(x, tab)
